# baseline (speedup 1.0000x reference)
import functools

import jax
import jax.numpy as jnp
from jax import lax
from jax.experimental import pallas as pl
from jax.experimental.pallas import tpu as pltpu
from jax.experimental.pallas import tpu_sc as plsc

LANES = 16
STRIP = 128
D_HALF = 1024
D_P = 128
BLK = 8
CHUNK = 256


def _splat(x):
    return jnp.full((LANES,), x, jnp.int32)


def _bcast_lane(v, a):
    return jnp.take_along_axis(v, _splat(a), axis=0, mode="promise_in_bounds")


def _sc_geometry():
    try:
        info = plsc.get_sparse_core_info()
        return info.num_cores, info.num_subcores
    except Exception:
        return 2, 16


def _nnue_body(n_cores, bags_per_worker, n_active, d_l1, d_p,
               wi, wv, wvp, bi, bv, bvp, w_l1, w_p, b_l1, b_p16,
               wp_out, bp_out, pw_out, pb_out,
               idx_blk, val_blk, valp_blk, buf_lo, buf_hi, buf_pa, buf_pb,
               obl1_a, obl1_b, pchunk, bias_v, bias_p,
               sem_lo, sem_hi, sem_pa, sem_pb, sem_oa, sem_ob):
    half = n_active // 2
    wid = lax.axis_index("s") * n_cores + lax.axis_index("c")
    base = wid * bags_per_worker
    last = bags_per_worker - 1
    p_bufs = ((buf_pa, sem_pa), (buf_pb, sem_pb))

    pltpu.sync_copy(b_l1, bias_v)
    pltpu.sync_copy(b_p16, bias_p)

    def idx_row(g):
        return idx_blk.at[pl.ds(g * n_active, n_active)]

    def gather_lo(g):
        return pltpu.make_async_copy(
            w_l1.at[idx_row(g), pl.ds(0, D_HALF // 2)], buf_lo, sem_lo)

    def gather_hi(g):
        return pltpu.make_async_copy(
            w_l1.at[idx_row(g), pl.ds(D_HALF // 2, D_HALF // 2)],
            buf_hi, sem_hi)

    def gather_p(g, buf_p, sem_p):
        return pltpu.make_async_copy(w_p.at[idx_row(g)], buf_p, sem_p)

    ev_idx = lax.iota(jnp.int32, LANES) * 2
    od_idx = ev_idx + 1
    n_ch = STRIP // (2 * LANES)

    def accumulate(buf, blk, k, d_off, width, vp0, vp1):
        GRP = 2

        def strip_body(s, _s):
            off = s * STRIP

            def group_body(grp, accs):
                zero = jnp.zeros((2 * LANES,), jnp.bfloat16)
                paccs = [zero] * n_ch
                for step in range(GRP):
                    a = grp * GRP + step
                    bc0b = plsc.bitcast(_bcast_lane(vp0, a), jnp.bfloat16)
                    bc1b = plsc.bitcast(_bcast_lane(vp1, a), jnp.bfloat16)
                    offw = off // 2
                    for ch in range(n_ch):
                        lo = plsc.bitcast(
                            buf[a, pl.ds(offw + ch * LANES, LANES)],
                            jnp.bfloat16)
                        hi = plsc.bitcast(
                            buf[a + half, pl.ds(offw + ch * LANES, LANES)],
                            jnp.bfloat16)
                        paccs[ch] = paccs[ch] + lo * bc0b + hi * bc1b
                new = list(accs)
                for ch in range(n_ch):
                    pe, po = plsc.unpack(
                        paccs[ch], format=plsc.PackFormat.INTERLEAVED)
                    new[2 * ch] = new[2 * ch] + pe
                    new[2 * ch + 1] = new[2 * ch + 1] + po
                return tuple(new)

            accs0 = tuple(
                bias_v[pl.ds(d_off + off + ch * 2 * LANES + h * LANES, LANES)]
                for ch in range(n_ch) for h in range(2)
            )
            accs = lax.fori_loop(0, half // GRP, group_body, accs0)
            for ch in range(n_ch):
                tgt = blk.at[k, pl.ds(d_off + off + ch * 2 * LANES, 2 * LANES)]
                plsc.store_scatter(tgt, [ev_idx], accs[2 * ch])
                plsc.store_scatter(tgt, [od_idx], accs[2 * ch + 1])
            return 0

        lax.fori_loop(0, width // STRIP, strip_body, 0)

    roll8 = (jnp.arange(LANES, dtype=jnp.int32) + 8) % LANES

    for idx_h, val_h, valp_h, out_h, pout_h in (
            (wi, wv, wvp, wp_out, pw_out), (bi, bv, bvp, bp_out, pb_out)):
        for c in range(bags_per_worker // CHUNK):
            cbase = base + c * CHUNK
            pltpu.sync_copy(
                idx_h.at[pl.ds(cbase * n_active, CHUNK * n_active)], idx_blk)
            pltpu.sync_copy(
                val_h.at[pl.ds(cbase * n_active, CHUNK * n_active)], val_blk)
            pltpu.sync_copy(
                valp_h.at[pl.ds(cbase * n_active, CHUNK * n_active)], valp_blk)

            gather_lo(0).start()
            gather_p(0, buf_pa, sem_pa).start()

            def super_body(i, _, cbase=cbase, out_h=out_h):
                for jb, (obl1, sem_o) in enumerate(
                        ((obl1_a, sem_oa), (obl1_b, sem_ob))):
                    b0 = i * (2 * BLK) + jb * BLK
                    row0 = cbase + b0

                    @pl.when(b0 >= 2 * BLK)
                    def _():
                        pltpu.make_async_copy(
                            obl1,
                            out_h.at[pl.ds(row0 - 2 * BLK, BLK),
                                     pl.ds(0, d_l1)],
                            sem_o).wait()

                    def pair_k(kp, _, b0=b0, obl1=obl1):
                      pacc_prev = [None]
                      for j in range(2):
                        k = kp * 2 + j
                        g = b0 + k
                        v0 = val_blk[pl.ds(g * n_active, LANES)]
                        v1 = val_blk[pl.ds(g * n_active + LANES, LANES)]
                        vp0 = valp_blk[pl.ds(g * n_active, LANES)]
                        vp1 = valp_blk[pl.ds(g * n_active + LANES, LANES)]

                        gather_hi(g).start()

                        @pl.when(g < CHUNK - 1)
                        def _(g=g, j=j):
                            buf_pn, sem_pn = p_bufs[(j + 1) % 2]
                            gather_p(g + 1, buf_pn, sem_pn).start()

                        gather_lo(g).wait()
                        accumulate(buf_lo, obl1, k, 0, D_HALF, vp0, vp1)

                        @pl.when(g < CHUNK - 1)
                        def _(g=g):
                            gather_lo(g + 1).start()

                        gather_hi(g).wait()
                        accumulate(buf_hi, obl1, k, D_HALF, D_HALF, vp0, vp1)

                        buf_p, sem_p = p_bufs[j % 2]
                        gather_p(g, buf_p, sem_p).wait()

                        def pinner(a, acc, buf_p=buf_p, v0=v0, v1=v1):
                            bc0 = _bcast_lane(v0, a)
                            bc1 = _bcast_lane(v1, a)
                            return (acc + bc0 * buf_p[a, pl.ds(0, LANES)]
                                    + bc1 * buf_p[a + half, pl.ds(0, LANES)])

                        pacc = lax.fori_loop(0, half, pinner, bias_p[...])
                        if j % 2 == 0:
                            pacc_prev[0] = pacc
                        else:
                            both = pacc_prev[0] + jnp.take_along_axis(
                                pacc, roll8, axis=0, mode="promise_in_bounds")
                            pchunk[pl.ds((g - 1) * d_p, LANES)] = both

                      return 0

                    lax.fori_loop(0, BLK // 2, pair_k, 0)
                    pltpu.async_copy(
                        obl1, out_h.at[pl.ds(row0, BLK), pl.ds(0, d_l1)],
                        sem_o)
                return 0

            lax.fori_loop(0, CHUNK // (2 * BLK), super_body, 0)

            pltpu.sync_copy(pchunk.at[pl.ds(0, CHUNK * d_p)],
                            pout_h.at[pl.ds(cbase * d_p, CHUNK * d_p)])
            for obl1, sem_o, nback in ((obl1_a, sem_oa, 2),
                                       (obl1_b, sem_ob, 1)):
                row0 = cbase + CHUNK - nback * BLK
                pltpu.make_async_copy(
                    obl1, out_h.at[pl.ds(row0, BLK), pl.ds(0, d_l1)],
                    sem_o).wait()


def kernel(white_indices, white_values, black_indices, black_values,
           W_l1, b_l1, W_psqt, b_psqt):
    batch, n_active = white_indices.shape
    n_feat, d_l1 = W_l1.shape
    d_p = W_psqt.shape[1]
    d_out = d_l1 + d_p
    assert d_l1 == 2 * D_HALF and d_p <= LANES

    n_cores, n_subcores = _sc_geometry()
    n_workers = n_cores * n_subcores
    assert batch % (n_workers * CHUNK) == 0
    bags_per_worker = batch // n_workers

    u = lax.bitcast_convert_type(W_l1, jnp.uint32)
    r = (u + jnp.uint32(0x7FFF) + ((u >> 16) & jnp.uint32(1))) >> 16
    rp = r.reshape(n_feat, d_l1 // 2, 2)
    w_bf = lax.bitcast_convert_type(rp[..., 0] | (rp[..., 1] << 16),
                                    jnp.float32)
    w_p = jnp.pad(W_psqt, ((0, 0), (0, D_P - d_p)))
    b_p16 = jnp.pad(b_psqt, (0, LANES - d_p))
    b_perm = b_l1.reshape(-1, LANES, 2).transpose(0, 2, 1).reshape(-1)

    def pack_vals(v):
        bits = lax.bitcast_convert_type(
            v.astype(jnp.bfloat16), jnp.uint16).astype(jnp.uint32)
        return (bits * jnp.uint32(0x10001)).reshape(-1)

    mesh = plsc.VectorSubcoreMesh(core_axis_name="c", subcore_axis_name="s",
                                  num_cores=n_cores, num_subcores=n_subcores)
    body = functools.partial(_nnue_body, n_cores, bags_per_worker, n_active,
                             d_l1, d_p)
    out = pl.kernel(
        body,
        out_type=(
            jax.ShapeDtypeStruct((batch, d_out), jnp.float32),
            jax.ShapeDtypeStruct((batch, d_out), jnp.float32),
            jax.ShapeDtypeStruct((batch * d_p,), jnp.float32),
            jax.ShapeDtypeStruct((batch * d_p,), jnp.float32),
        ),
        mesh=mesh,
        compiler_params=pltpu.CompilerParams(needs_layout_passes=False),
        scratch_types=[
            pltpu.VMEM((CHUNK * n_active,), jnp.int32),
            pltpu.VMEM((CHUNK * n_active,), jnp.float32),
            pltpu.VMEM((CHUNK * n_active,), jnp.uint32),
            pltpu.VMEM((n_active, D_HALF // 2), jnp.float32),
            pltpu.VMEM((n_active, D_HALF // 2), jnp.float32),
            pltpu.VMEM((n_active, D_P), jnp.float32),
            pltpu.VMEM((n_active, D_P), jnp.float32),
            pltpu.VMEM((BLK, d_l1), jnp.float32),
            pltpu.VMEM((BLK, d_l1), jnp.float32),
            pltpu.VMEM((CHUNK * W_psqt.shape[1] + 8,), jnp.float32),
            pltpu.VMEM((d_l1,), jnp.float32),
            pltpu.VMEM((LANES,), jnp.float32),
            pltpu.SemaphoreType.DMA,
            pltpu.SemaphoreType.DMA,
            pltpu.SemaphoreType.DMA,
            pltpu.SemaphoreType.DMA,
            pltpu.SemaphoreType.DMA,
            pltpu.SemaphoreType.DMA,
        ],
    )(white_indices.reshape(-1), white_values.reshape(-1),
      pack_vals(white_values),
      black_indices.reshape(-1), black_values.reshape(-1),
      pack_vals(black_values),
      w_bf, w_p, b_perm, b_p16)
    wp = lax.dynamic_update_slice(out[0], out[2].reshape(batch, d_p), (0, d_l1))
    bp = lax.dynamic_update_slice(out[1], out[3].reshape(batch, d_p), (0, d_l1))
    return wp, bp


# speedup vs baseline: 1.7112x; 1.7112x over previous
import functools

import jax
import jax.numpy as jnp
from jax import lax
from jax.experimental import pallas as pl
from jax.experimental.pallas import tpu as pltpu
from jax.experimental.pallas import tpu_sc as plsc

LANES = 16
STRIP = 128
D_HALF = 1024
D_P = 128
BLK = 8
CHUNK = 256


def _splat(x):
    return jnp.full((LANES,), x, jnp.int32)


def _bcast_lane(v, a):
    return jnp.take_along_axis(v, _splat(a), axis=0, mode="promise_in_bounds")


def _sc_geometry():
    try:
        info = plsc.get_sparse_core_info()
        return info.num_cores, info.num_subcores
    except Exception:
        return 2, 16


def _nnue_body(n_cores, bags_per_worker, n_active, d_l1, d_p,
               wi, wv, wvp, bi, bv, bvp, w_l1, w_p, b_l1, b_p16,
               wp_out, bp_out, pw_out, pb_out,
               idx_blk, val_blk, valp_blk, buf_lo, buf_hi, buf_pa, buf_pb,
               obl1_a, obl1_b, pchunk, bias_v, bias_p,
               sem_lo, sem_hi, sem_pa, sem_pb, sem_oa, sem_ob):
    half = n_active // 2
    wid = lax.axis_index("s") * n_cores + lax.axis_index("c")
    base = wid * bags_per_worker
    last = bags_per_worker - 1
    p_bufs = ((buf_pa, sem_pa), (buf_pb, sem_pb))

    pltpu.sync_copy(b_l1, bias_v)
    pltpu.sync_copy(b_p16, bias_p)

    def idx_row(g):
        return idx_blk.at[pl.ds(g * n_active, n_active)]

    def gather_lo(g):
        return pltpu.make_async_copy(
            w_l1.at[idx_row(g), pl.ds(0, D_HALF // 2)], buf_lo, sem_lo)

    def gather_hi(g):
        return pltpu.make_async_copy(
            w_l1.at[idx_row(g), pl.ds(D_HALF // 2, D_HALF // 2)],
            buf_hi, sem_hi)

    def gather_p(g, buf_p, sem_p):
        return pltpu.make_async_copy(w_p.at[idx_row(g)], buf_p, sem_p)

    n_ch = STRIP // (2 * LANES)

    def accumulate(buf, blk, k, woff, vp0, vp1):
        GRP = 2
        width_w = D_HALF // 2

        def strip_body(s, _s):
            offw = woff + s * n_ch * LANES

            def group_body(grp, accs):
                zero = jnp.zeros((2 * LANES,), jnp.bfloat16)
                paccs = [zero] * n_ch
                for step in range(GRP):
                    a = grp * GRP + step
                    bc0b = plsc.bitcast(_bcast_lane(vp0, a), jnp.bfloat16)
                    bc1b = plsc.bitcast(_bcast_lane(vp1, a), jnp.bfloat16)
                    for ch in range(n_ch):
                        lo = plsc.bitcast(
                            buf[a, pl.ds(offw - woff + ch * LANES, LANES)],
                            jnp.bfloat16)
                        hi = plsc.bitcast(
                            buf[a + half,
                                pl.ds(offw - woff + ch * LANES, LANES)],
                            jnp.bfloat16)
                        paccs[ch] = paccs[ch] + lo * bc0b + hi * bc1b
                new = list(accs)
                for ch in range(n_ch):
                    pe, po = plsc.unpack(
                        paccs[ch], format=plsc.PackFormat.INTERLEAVED)
                    new[2 * ch] = new[2 * ch] + pe
                    new[2 * ch + 1] = new[2 * ch + 1] + po
                return tuple(new)

            accs0 = tuple(
                bias_v[pl.ds(offw + ch * LANES + h * D_HALF, LANES)]
                for ch in range(n_ch) for h in range(2)
            )
            accs = lax.fori_loop(0, half // GRP, group_body, accs0)
            for ch in range(n_ch):
                col = offw + ch * LANES
                blk[k, pl.ds(col, LANES)] = accs[2 * ch]
                blk[k, pl.ds(col + D_HALF, LANES)] = accs[2 * ch + 1]
            return 0

        lax.fori_loop(0, width_w // (n_ch * LANES), strip_body, 0)

    roll8 = (jnp.arange(LANES, dtype=jnp.int32) + 8) % LANES

    for idx_h, val_h, valp_h, out_h, pout_h in (
            (wi, wv, wvp, wp_out, pw_out), (bi, bv, bvp, bp_out, pb_out)):
        for c in range(bags_per_worker // CHUNK):
            cbase = base + c * CHUNK
            pltpu.sync_copy(
                idx_h.at[pl.ds(cbase * n_active, CHUNK * n_active)], idx_blk)
            pltpu.sync_copy(
                val_h.at[pl.ds(cbase * n_active, CHUNK * n_active)], val_blk)
            pltpu.sync_copy(
                valp_h.at[pl.ds(cbase * n_active, CHUNK * n_active)], valp_blk)

            gather_lo(0).start()
            gather_p(0, buf_pa, sem_pa).start()

            def super_body(i, _, cbase=cbase, out_h=out_h):
                for jb, (obl1, sem_o) in enumerate(
                        ((obl1_a, sem_oa), (obl1_b, sem_ob))):
                    b0 = i * (2 * BLK) + jb * BLK
                    row0 = cbase + b0

                    @pl.when(b0 >= 2 * BLK)
                    def _():
                        pltpu.make_async_copy(
                            obl1,
                            out_h.at[pl.ds(row0 - 2 * BLK, BLK),
                                     pl.ds(0, d_l1)],
                            sem_o).wait()

                    def pair_k(kp, _, b0=b0, obl1=obl1):
                      pacc_prev = [None]
                      for j in range(2):
                        k = kp * 2 + j
                        g = b0 + k
                        v0 = val_blk[pl.ds(g * n_active, LANES)]
                        v1 = val_blk[pl.ds(g * n_active + LANES, LANES)]
                        vp0 = valp_blk[pl.ds(g * n_active, LANES)]
                        vp1 = valp_blk[pl.ds(g * n_active + LANES, LANES)]

                        gather_hi(g).start()

                        @pl.when(g < CHUNK - 1)
                        def _(g=g, j=j):
                            buf_pn, sem_pn = p_bufs[(j + 1) % 2]
                            gather_p(g + 1, buf_pn, sem_pn).start()

                        gather_lo(g).wait()
                        accumulate(buf_lo, obl1, k, 0, vp0, vp1)

                        @pl.when(g < CHUNK - 1)
                        def _(g=g):
                            gather_lo(g + 1).start()

                        gather_hi(g).wait()
                        accumulate(buf_hi, obl1, k, D_HALF // 2, vp0, vp1)

                        buf_p, sem_p = p_bufs[j % 2]
                        gather_p(g, buf_p, sem_p).wait()

                        def pinner(a, acc, buf_p=buf_p, v0=v0, v1=v1):
                            bc0 = _bcast_lane(v0, a)
                            bc1 = _bcast_lane(v1, a)
                            return (acc + bc0 * buf_p[a, pl.ds(0, LANES)]
                                    + bc1 * buf_p[a + half, pl.ds(0, LANES)])

                        pacc = lax.fori_loop(0, half, pinner, bias_p[...])
                        if j % 2 == 0:
                            pacc_prev[0] = pacc
                        else:
                            both = pacc_prev[0] + jnp.take_along_axis(
                                pacc, roll8, axis=0, mode="promise_in_bounds")
                            pchunk[pl.ds((g - 1) * d_p, LANES)] = both

                      return 0

                    lax.fori_loop(0, BLK // 2, pair_k, 0)
                    pltpu.async_copy(
                        obl1, out_h.at[pl.ds(row0, BLK), pl.ds(0, d_l1)],
                        sem_o)
                return 0

            lax.fori_loop(0, CHUNK // (2 * BLK), super_body, 0)

            pltpu.sync_copy(pchunk.at[pl.ds(0, CHUNK * d_p)],
                            pout_h.at[pl.ds(cbase * d_p, CHUNK * d_p)])
            for obl1, sem_o, nback in ((obl1_a, sem_oa, 2),
                                       (obl1_b, sem_ob, 1)):
                row0 = cbase + CHUNK - nback * BLK
                pltpu.make_async_copy(
                    obl1, out_h.at[pl.ds(row0, BLK), pl.ds(0, d_l1)],
                    sem_o).wait()


def kernel(white_indices, white_values, black_indices, black_values,
           W_l1, b_l1, W_psqt, b_psqt):
    batch, n_active = white_indices.shape
    n_feat, d_l1 = W_l1.shape
    d_p = W_psqt.shape[1]
    d_out = d_l1 + d_p
    assert d_l1 == 2 * D_HALF and d_p <= LANES

    n_cores, n_subcores = _sc_geometry()
    n_workers = n_cores * n_subcores
    assert batch % (n_workers * CHUNK) == 0
    bags_per_worker = batch // n_workers

    u = lax.bitcast_convert_type(W_l1, jnp.uint32)
    r = (u + jnp.uint32(0x7FFF) + ((u >> 16) & jnp.uint32(1))) >> 16
    w_bf = lax.bitcast_convert_type(
        r[:, :d_l1 // 2] | (r[:, d_l1 // 2:] << 16), jnp.float32)
    w_p = jnp.pad(W_psqt, ((0, 0), (0, D_P - d_p)))
    b_p16 = jnp.pad(b_psqt, (0, LANES - d_p))
    def pack_vals(v):
        bits = lax.bitcast_convert_type(
            v.astype(jnp.bfloat16), jnp.uint16).astype(jnp.uint32)
        return (bits * jnp.uint32(0x10001)).reshape(-1)

    mesh = plsc.VectorSubcoreMesh(core_axis_name="c", subcore_axis_name="s",
                                  num_cores=n_cores, num_subcores=n_subcores)
    body = functools.partial(_nnue_body, n_cores, bags_per_worker, n_active,
                             d_l1, d_p)
    out = pl.kernel(
        body,
        out_type=(
            jax.ShapeDtypeStruct((batch, d_out), jnp.float32),
            jax.ShapeDtypeStruct((batch, d_out), jnp.float32),
            jax.ShapeDtypeStruct((batch * d_p,), jnp.float32),
            jax.ShapeDtypeStruct((batch * d_p,), jnp.float32),
        ),
        mesh=mesh,
        compiler_params=pltpu.CompilerParams(needs_layout_passes=False),
        scratch_types=[
            pltpu.VMEM((CHUNK * n_active,), jnp.int32),
            pltpu.VMEM((CHUNK * n_active,), jnp.float32),
            pltpu.VMEM((CHUNK * n_active,), jnp.uint32),
            pltpu.VMEM((n_active, D_HALF // 2), jnp.float32),
            pltpu.VMEM((n_active, D_HALF // 2), jnp.float32),
            pltpu.VMEM((n_active, D_P), jnp.float32),
            pltpu.VMEM((n_active, D_P), jnp.float32),
            pltpu.VMEM((BLK, d_l1), jnp.float32),
            pltpu.VMEM((BLK, d_l1), jnp.float32),
            pltpu.VMEM((CHUNK * W_psqt.shape[1] + 8,), jnp.float32),
            pltpu.VMEM((d_l1,), jnp.float32),
            pltpu.VMEM((LANES,), jnp.float32),
            pltpu.SemaphoreType.DMA,
            pltpu.SemaphoreType.DMA,
            pltpu.SemaphoreType.DMA,
            pltpu.SemaphoreType.DMA,
            pltpu.SemaphoreType.DMA,
            pltpu.SemaphoreType.DMA,
        ],
    )(white_indices.reshape(-1), white_values.reshape(-1),
      pack_vals(white_values),
      black_indices.reshape(-1), black_values.reshape(-1),
      pack_vals(black_values),
      w_bf, w_p, b_l1, b_p16)
    wp = lax.dynamic_update_slice(out[0], out[2].reshape(batch, d_p), (0, d_l1))
    bp = lax.dynamic_update_slice(out[1], out[3].reshape(batch, d_p), (0, d_l1))
    return wp, bp
